# baseline (speedup 1.0000x reference)
import jax
import jax.numpy as jnp
from jax import lax
from jax.experimental import pallas as pl
from jax.experimental.pallas import tpu as pltpu
from jax.experimental.pallas import tpu_sc as plsc

_G = 1024
_NC, _NS = 2, 16
_NW = _NC * _NS
_R = 160
_U = 16
_RG = 4
_CG = 8
_DC = 64


def _logits_stats_body(x_ref, w_ref, o_ref, mz_ref, acc):
    i = pl.program_id(0)

    @pl.when(i == 0)
    def _():
        acc[0] = -jnp.inf
        acc[1] = 0.0

    l = jnp.dot(x_ref[...], w_ref[...], preferred_element_type=jnp.float32)
    o_ref[...] = l
    m0 = acc[0]
    m1 = jnp.maximum(m0, jnp.max(l))
    acc[1] = acc[1] * jnp.exp(m0 - m1) + jnp.sum(jnp.exp(l - m1))
    acc[0] = m1

    @pl.when(i == pl.num_programs(0) - 1)
    def _():
        mz_ref[...] = jnp.stack([acc[0], acc[1]]).reshape(1, 2)


def _make_pool(n, d, npad):
    rpt = npad // _RG
    nch = rpt // _R

    def body(x_hbm, a16_hbm, b16_hbm, zeros_hbm, part_hbm,
             accum, xb0, xb1, ab0, ab1, ib0, ib1, sem0, sem1):
        c = lax.axis_index("c")
        s = lax.axis_index("s")
        wid = s * _NC + c
        cg = wid // _RG
        rg = wid % _RG
        base = rg * rpt
        a0 = (cg // 2) * 128
        off = (cg % 2) * _DC

        xb, ab, ib = [xb0, xb1], [ab0, ab1], [ib0, ib1]
        sems = [sem0, sem1]

        def fire(k, b):
            start = base + k * _R
            xs = jnp.minimum(start, n - _R)
            pltpu.async_copy(
                x_hbm.at[pl.ds(xs, _R), pl.ds(a0, 128)], xb[b], sems[b])
            pltpu.async_copy(
                a16_hbm.at[pl.ds(start * 16, _R * 16)], ab[b], sems[b])
            pltpu.async_copy(
                b16_hbm.at[pl.ds(start * 16, _R * 16)], ib[b], sems[b])

        def drain(b):
            pltpu.make_async_copy(
                x_hbm.at[pl.ds(0, _R), pl.ds(a0, 128)], xb[b], sems[b]).wait()
            pltpu.make_async_copy(
                a16_hbm.at[pl.ds(0, _R * 16)], ab[b], sems[b]).wait()
            pltpu.make_async_copy(
                b16_hbm.at[pl.ds(0, _R * 16)], ib[b], sems[b]).wait()

        pltpu.sync_copy(zeros_hbm, accum)

        fire(0, 0)

        half = _R // 2

        def outer(g, carry):
            for b in range(2):
                k = 2 * g + b
                fire(jnp.minimum(k + 1, nch - 1), 1 - b)
                drain(b)

                def rows(rb, carry2):
                    c1 = list(carry2[0:4])
                    c2 = list(carry2[4:8])
                    fp1, fp2 = carry2[8], carry2[9]
                    for u in range(_U // 2):
                        r1 = rb * (_U // 2) + u
                        r2 = r1 + half
                        a1 = ab[b][pl.ds(r1 * 16, 16)]
                        a2 = ab[b][pl.ds(r2 * 16, 16)]
                        f1 = ib[b][pl.ds(r1 * 16, 16)]
                        f2 = ib[b][pl.ds(r2 * 16, 16)]
                        ch1 = f1 != fp1
                        ch2 = f2 != fp2
                        for q in range(_DC // 16):
                            plsc.addupdate_scatter(
                                accum, [fp1 + q * 16], c1[q], mask=ch1)
                            v1 = xb[b][r1, pl.ds(off + q * 16, 16)] * a1
                            c1[q] = jnp.where(ch1, 0.0, c1[q]) + v1
                            plsc.addupdate_scatter(
                                accum, [fp2 + q * 16], c2[q], mask=ch2)
                            v2 = xb[b][r2, pl.ds(off + q * 16, 16)] * a2
                            c2[q] = jnp.where(ch2, 0.0, c2[q]) + v2
                        fp1, fp2 = f1, f2
                    return (*c1, *c2, fp1, fp2)

                carry = lax.fori_loop(0, half // (_U // 2), rows, carry)
            return carry

        z16 = jnp.zeros((16,), jnp.float32)
        cols = lax.iota(jnp.int32, 16)
        fin = lax.fori_loop(0, nch // 2, outer, (z16,) * 8 + (cols, cols))
        drain(0)
        for q in range(_DC // 16):
            plsc.addupdate_scatter(accum, [fin[8] + q * 16], fin[q])
            plsc.addupdate_scatter(accum, [fin[9] + q * 16], fin[4 + q])
        pltpu.sync_copy(accum, part_hbm.at[wid])

    return pl.kernel(
        body,
        out_type=jax.ShapeDtypeStruct((_NW, _G * _DC), jnp.float32),
        mesh=plsc.VectorSubcoreMesh(core_axis_name="c", subcore_axis_name="s"),
        compiler_params=pltpu.CompilerParams(needs_layout_passes=False),
        scratch_types=[
            pltpu.VMEM((_G * _DC,), jnp.float32),
            pltpu.VMEM((_R, 128), jnp.float32),
            pltpu.VMEM((_R, 128), jnp.float32),
            pltpu.VMEM((_R * 16,), jnp.float32),
            pltpu.VMEM((_R * 16,), jnp.float32),
            pltpu.VMEM((_R * 16,), jnp.int32),
            pltpu.VMEM((_R * 16,), jnp.int32),
            pltpu.SemaphoreType.DMA,
            pltpu.SemaphoreType.DMA,
        ],
    )


def _combine_body(p_ref, o_ref):
    p = p_ref[...]
    s0 = p[0, 0] + p[0, 1] + p[0, 2] + p[0, 3]
    s1 = p[1, 0] + p[1, 1] + p[1, 2] + p[1, 3]
    o_ref[...] = jnp.concatenate([s0, s1], axis=1)


def kernel(x, batch, W, b):
    n, d = x.shape
    npad = _NW * _R * (-(-n // (_NW * _R)))

    br = 2000
    logits, mz = pl.pallas_call(
        _logits_stats_body,
        grid=(n // br,),
        in_specs=[pl.BlockSpec((br, d), lambda i: (i, 0)),
                  pl.BlockSpec((d, 1), lambda i: (0, 0))],
        out_specs=[pl.BlockSpec((br, 1), lambda i: (i, 0)),
                   pl.BlockSpec((1, 2), lambda i: (0, 0))],
        out_shape=[jax.ShapeDtypeStruct((n, 1), jnp.float32),
                   jax.ShapeDtypeStruct((1, 2), jnp.float32)],
        scratch_shapes=[pltpu.SMEM((2,), jnp.float32)],
    )(x, W.reshape(d, 1))

    bs = 800
    nvb = n // bs

    def _alpha_body(l_ref, b_ref, mz_ref, a_ref, f_ref):
        i = pl.program_id(0)
        m = mz_ref[0, 0]
        z = mz_ref[0, 1]
        valid = (lax.broadcasted_iota(jnp.int32, (bs, 1), 0) + i * bs) < n
        a = jnp.where(valid, jnp.exp(l_ref[...] - m) / z, 0.0)
        a_ref[...] = jnp.broadcast_to(a, (bs, 16))
        fb = jnp.where(valid, b_ref[...], 0) * _DC
        f_ref[...] = fb + lax.broadcasted_iota(jnp.int32, (bs, 16), 1)

    a16, b16 = pl.pallas_call(
        _alpha_body,
        grid=(npad // bs,),
        in_specs=[
            pl.BlockSpec((bs, 1), lambda i: (jnp.minimum(i, nvb - 1), 0)),
            pl.BlockSpec((bs, 1), lambda i: (jnp.minimum(i, nvb - 1), 0)),
            pl.BlockSpec((1, 2), lambda i: (0, 0))],
        out_specs=[pl.BlockSpec((bs, 16), lambda i: (i, 0)),
                   pl.BlockSpec((bs, 16), lambda i: (i, 0))],
        out_shape=[jax.ShapeDtypeStruct((npad, 16), jnp.float32),
                   jax.ShapeDtypeStruct((npad, 16), jnp.int32)],
    )(logits, batch.astype(jnp.int32).reshape(n, 1), mz)

    zeros = jnp.zeros((_G * _DC,), jnp.float32)

    part = _make_pool(n, d, npad)(
        x, a16.reshape(npad * 16), b16.reshape(npad * 16), zeros)

    out = pl.pallas_call(
        _combine_body,
        grid=(_CG // 2,),
        in_specs=[pl.BlockSpec((2, _RG, _G, _DC), lambda i: (i, 0, 0, 0))],
        out_specs=pl.BlockSpec((_G, 2 * _DC), lambda i: (0, i)),
        out_shape=jax.ShapeDtypeStruct((_G, d), jnp.float32),
    )(part.reshape(_CG, _RG, _G, _DC))

    alpha = a16[:n, :1]
    return (out, alpha)


# speedup vs baseline: 1.0260x; 1.0260x over previous
import jax
import jax.numpy as jnp
from jax import lax
from jax.experimental import pallas as pl
from jax.experimental.pallas import tpu as pltpu
from jax.experimental.pallas import tpu_sc as plsc

_G = 1024
_NC, _NS = 2, 16
_NW = _NC * _NS
_R = 160
_U = 16
_RG = 4
_CG = 8
_DC = 64


def _logits_stats_body(x_ref, w_ref, o_ref, mz_ref, acc):
    i = pl.program_id(0)

    @pl.when(i == 0)
    def _():
        acc[0] = -jnp.inf
        acc[1] = 0.0

    l = jnp.dot(x_ref[...], w_ref[...], preferred_element_type=jnp.float32)
    o_ref[...] = l
    m0 = acc[0]
    m1 = jnp.maximum(m0, jnp.max(l))
    acc[1] = acc[1] * jnp.exp(m0 - m1) + jnp.sum(jnp.exp(l - m1))
    acc[0] = m1

    @pl.when(i == pl.num_programs(0) - 1)
    def _():
        mz_ref[...] = jnp.stack([acc[0], acc[1]]).reshape(1, 2)


def _make_pool(n, d, npad):
    rpt = npad // _RG
    nch = rpt // _R

    def body(x_hbm, a16_hbm, b16_hbm, zeros_hbm, part_hbm,
             accum, xb0, xb1, ab0, ab1, ib0, ib1, sem0, sem1):
        c = lax.axis_index("c")
        s = lax.axis_index("s")
        wid = s * _NC + c
        cg = wid // _RG
        rg = wid % _RG
        base = rg * rpt
        a0 = (cg // 2) * 128
        off = (cg % 2) * _DC

        xb, ab, ib = [xb0, xb1], [ab0, ab1], [ib0, ib1]
        sems = [sem0, sem1]

        def fire(k, b):
            start = base + k * _R
            xs = jnp.minimum(start, n - _R)
            pltpu.async_copy(
                x_hbm.at[pl.ds(xs, _R), pl.ds(a0, 128)], xb[b], sems[b])
            pltpu.async_copy(
                a16_hbm.at[pl.ds(start * 16, _R * 16)], ab[b], sems[b])
            pltpu.async_copy(
                b16_hbm.at[pl.ds(start * 16, _R * 16)], ib[b], sems[b])

        def drain(b):
            pltpu.make_async_copy(
                x_hbm.at[pl.ds(0, _R), pl.ds(a0, 128)], xb[b], sems[b]).wait()
            pltpu.make_async_copy(
                a16_hbm.at[pl.ds(0, _R * 16)], ab[b], sems[b]).wait()
            pltpu.make_async_copy(
                b16_hbm.at[pl.ds(0, _R * 16)], ib[b], sems[b]).wait()

        pltpu.sync_copy(zeros_hbm, accum)

        fire(0, 0)

        half = _R // 2

        def outer(g, carry):
            for b in range(2):
                k = 2 * g + b
                fire(jnp.minimum(k + 1, nch - 1), 1 - b)
                drain(b)

                def rows(rb, carry2):
                    c1 = list(carry2[0:4])
                    c2 = list(carry2[4:8])
                    fp1, fp2 = carry2[8], carry2[9]
                    for u in range(_U // 2):
                        r1 = rb * (_U // 2) + u
                        r2 = r1 + half
                        a1 = ab[b][pl.ds(r1 * 16, 16)]
                        a2 = ab[b][pl.ds(r2 * 16, 16)]
                        f1 = ib[b][pl.ds(r1 * 16, 16)]
                        f2 = ib[b][pl.ds(r2 * 16, 16)]
                        ch1 = f1 != fp1
                        ch2 = f2 != fp2
                        for q in range(_DC // 16):
                            plsc.addupdate_scatter(
                                accum, [fp1 + q * 16], c1[q], mask=ch1)
                            v1 = xb[b][r1, pl.ds(off + q * 16, 16)] * a1
                            c1[q] = jnp.where(ch1, 0.0, c1[q]) + v1
                            plsc.addupdate_scatter(
                                accum, [fp2 + q * 16], c2[q], mask=ch2)
                            v2 = xb[b][r2, pl.ds(off + q * 16, 16)] * a2
                            c2[q] = jnp.where(ch2, 0.0, c2[q]) + v2
                        fp1, fp2 = f1, f2
                    return (*c1, *c2, fp1, fp2)

                carry = lax.fori_loop(0, half // (_U // 2), rows, carry)
            return carry

        z16 = jnp.zeros((16,), jnp.float32)
        cols = lax.iota(jnp.int32, 16)
        fin = lax.fori_loop(0, nch // 2, outer, (z16,) * 8 + (cols, cols))
        drain(0)
        for q in range(_DC // 16):
            plsc.addupdate_scatter(accum, [fin[8] + q * 16], fin[q])
            plsc.addupdate_scatter(accum, [fin[9] + q * 16], fin[4 + q])
        pltpu.sync_copy(accum, part_hbm.at[wid])

    return pl.kernel(
        body,
        out_type=jax.ShapeDtypeStruct((_NW, _G * _DC), jnp.float32),
        mesh=plsc.VectorSubcoreMesh(core_axis_name="c", subcore_axis_name="s"),
        compiler_params=pltpu.CompilerParams(needs_layout_passes=False),
        scratch_types=[
            pltpu.VMEM((_G * _DC,), jnp.float32),
            pltpu.VMEM((_R, 128), jnp.float32),
            pltpu.VMEM((_R, 128), jnp.float32),
            pltpu.VMEM((_R * 16,), jnp.float32),
            pltpu.VMEM((_R * 16,), jnp.float32),
            pltpu.VMEM((_R * 16,), jnp.int32),
            pltpu.VMEM((_R * 16,), jnp.int32),
            pltpu.SemaphoreType.DMA,
            pltpu.SemaphoreType.DMA,
        ],
    )


def _combine_body(p_ref, o_ref):
    p = p_ref[...]
    s0 = p[0, 0] + p[0, 1] + p[0, 2] + p[0, 3]
    s1 = p[1, 0] + p[1, 1] + p[1, 2] + p[1, 3]
    o_ref[...] = jnp.concatenate([s0, s1], axis=1)


def kernel(x, batch, W, b):
    n, d = x.shape
    npad = _NW * _R * (-(-n // (_NW * _R)))

    br = 4000
    logits, mz = pl.pallas_call(
        _logits_stats_body,
        grid=(n // br,),
        in_specs=[pl.BlockSpec((br, d), lambda i: (i, 0)),
                  pl.BlockSpec((d, 1), lambda i: (0, 0))],
        out_specs=[pl.BlockSpec((br, 1), lambda i: (i, 0)),
                   pl.BlockSpec((1, 2), lambda i: (0, 0))],
        out_shape=[jax.ShapeDtypeStruct((n, 1), jnp.float32),
                   jax.ShapeDtypeStruct((1, 2), jnp.float32)],
        scratch_shapes=[pltpu.SMEM((2,), jnp.float32)],
    )(x, W.reshape(d, 1))

    bs = 800
    nvb = n // bs

    def _alpha_body(l_ref, b_ref, mz_ref, a_ref, f_ref):
        i = pl.program_id(0)
        m = mz_ref[0, 0]
        z = mz_ref[0, 1]
        valid = (lax.broadcasted_iota(jnp.int32, (bs, 1), 0) + i * bs) < n
        a = jnp.where(valid, jnp.exp(l_ref[...] - m) / z, 0.0)
        a_ref[...] = jnp.broadcast_to(a, (bs, 16))
        fb = jnp.where(valid, b_ref[...], 0) * _DC
        f_ref[...] = fb + lax.broadcasted_iota(jnp.int32, (bs, 16), 1)

    a16, b16 = pl.pallas_call(
        _alpha_body,
        grid=(npad // bs,),
        in_specs=[
            pl.BlockSpec((bs, 1), lambda i: (jnp.minimum(i, nvb - 1), 0)),
            pl.BlockSpec((bs, 1), lambda i: (jnp.minimum(i, nvb - 1), 0)),
            pl.BlockSpec((1, 2), lambda i: (0, 0))],
        out_specs=[pl.BlockSpec((bs, 16), lambda i: (i, 0)),
                   pl.BlockSpec((bs, 16), lambda i: (i, 0))],
        out_shape=[jax.ShapeDtypeStruct((npad, 16), jnp.float32),
                   jax.ShapeDtypeStruct((npad, 16), jnp.int32)],
    )(logits, batch.astype(jnp.int32).reshape(n, 1), mz)

    zeros = jnp.zeros((_G * _DC,), jnp.float32)

    part = _make_pool(n, d, npad)(
        x, a16.reshape(npad * 16), b16.reshape(npad * 16), zeros)

    out = pl.pallas_call(
        _combine_body,
        grid=(_CG // 2,),
        in_specs=[pl.BlockSpec((2, _RG, _G, _DC), lambda i: (i, 0, 0, 0))],
        out_specs=pl.BlockSpec((_G, 2 * _DC), lambda i: (0, i)),
        out_shape=jax.ShapeDtypeStruct((_G, d), jnp.float32),
    )(part.reshape(_CG, _RG, _G, _DC))

    alpha = a16[:n, :1]
    return (out, alpha)


# speedup vs baseline: 1.0297x; 1.0036x over previous
import jax
import jax.numpy as jnp
from jax import lax
from jax.experimental import pallas as pl
from jax.experimental.pallas import tpu as pltpu
from jax.experimental.pallas import tpu_sc as plsc

_G = 1024
_NC, _NS = 2, 16
_NW = _NC * _NS
_R = 160
_U = 16
_RG = 4
_CG = 8
_DC = 64


def _logits_stats_body(x_ref, w_ref, o_ref, mz_ref, acc):
    i = pl.program_id(0)

    @pl.when(i == 0)
    def _():
        acc[0] = -jnp.inf
        acc[1] = 0.0

    l = jnp.dot(x_ref[...], w_ref[...], preferred_element_type=jnp.float32)
    o_ref[...] = l
    m0 = acc[0]
    m1 = jnp.maximum(m0, jnp.max(l))
    acc[1] = acc[1] * jnp.exp(m0 - m1) + jnp.sum(jnp.exp(l - m1))
    acc[0] = m1

    @pl.when(i == pl.num_programs(0) - 1)
    def _():
        mz_ref[...] = jnp.stack([acc[0], acc[1]]).reshape(1, 2)


def _make_pool(n, d, npad):
    rpt = npad // _RG
    nch = rpt // _R

    def body(x_hbm, a16_hbm, b16_hbm, zeros_hbm, part_hbm,
             accum, xb0, xb1, ab0, ab1, ib0, ib1, sem0, sem1):
        c = lax.axis_index("c")
        s = lax.axis_index("s")
        wid = s * _NC + c
        cg = wid // _RG
        rg = wid % _RG
        base = rg * rpt
        a0 = (cg // 2) * 128
        off = (cg % 2) * _DC

        xb, ab, ib = [xb0, xb1], [ab0, ab1], [ib0, ib1]
        sems = [sem0, sem1]

        def fire(k, b):
            start = base + k * _R
            xs = jnp.minimum(start, n - _R)
            pltpu.async_copy(
                x_hbm.at[pl.ds(xs, _R), pl.ds(a0, 128)], xb[b], sems[b])
            pltpu.async_copy(
                a16_hbm.at[pl.ds(start * 16, _R * 16)], ab[b], sems[b])
            pltpu.async_copy(
                b16_hbm.at[pl.ds(start * 16, _R * 16)], ib[b], sems[b])

        def drain(b):
            pltpu.make_async_copy(
                x_hbm.at[pl.ds(0, _R), pl.ds(a0, 128)], xb[b], sems[b]).wait()
            pltpu.make_async_copy(
                a16_hbm.at[pl.ds(0, _R * 16)], ab[b], sems[b]).wait()
            pltpu.make_async_copy(
                b16_hbm.at[pl.ds(0, _R * 16)], ib[b], sems[b]).wait()

        pltpu.sync_copy(zeros_hbm, accum)

        fire(0, 0)

        half = _R // 2

        def outer(g, carry):
            for b in range(2):
                k = 2 * g + b
                fire(jnp.minimum(k + 1, nch - 1), 1 - b)
                drain(b)

                def rows(rb, carry2):
                    c1 = list(carry2[0:4])
                    c2 = list(carry2[4:8])
                    fp1, fp2 = carry2[8], carry2[9]
                    for u in range(_U // 2):
                        r1 = rb * (_U // 2) + u
                        r2 = r1 + half
                        a1 = ab[b][pl.ds(r1 * 16, 16)]
                        a2 = ab[b][pl.ds(r2 * 16, 16)]
                        f1 = ib[b][pl.ds(r1 * 16, 16)]
                        f2 = ib[b][pl.ds(r2 * 16, 16)]
                        ch1 = f1 != fp1
                        ch2 = f2 != fp2
                        for q in range(_DC // 16):
                            plsc.addupdate_scatter(
                                accum, [fp1 + q * 16], c1[q], mask=ch1)
                            v1 = xb[b][r1, pl.ds(off + q * 16, 16)] * a1
                            c1[q] = jnp.where(ch1, 0.0, c1[q]) + v1
                            plsc.addupdate_scatter(
                                accum, [fp2 + q * 16], c2[q], mask=ch2)
                            v2 = xb[b][r2, pl.ds(off + q * 16, 16)] * a2
                            c2[q] = jnp.where(ch2, 0.0, c2[q]) + v2
                        fp1, fp2 = f1, f2
                    return (*c1, *c2, fp1, fp2)

                carry = lax.fori_loop(0, half // (_U // 2), rows, carry)
            return carry

        z16 = jnp.zeros((16,), jnp.float32)
        cols = lax.iota(jnp.int32, 16)
        fin = lax.fori_loop(0, nch // 2, outer, (z16,) * 8 + (cols, cols))
        drain(0)
        for q in range(_DC // 16):
            plsc.addupdate_scatter(accum, [fin[8] + q * 16], fin[q])
            plsc.addupdate_scatter(accum, [fin[9] + q * 16], fin[4 + q])
        pltpu.sync_copy(accum, part_hbm.at[wid])

    return pl.kernel(
        body,
        out_type=jax.ShapeDtypeStruct((_NW, _G * _DC), jnp.float32),
        mesh=plsc.VectorSubcoreMesh(core_axis_name="c", subcore_axis_name="s"),
        compiler_params=pltpu.CompilerParams(needs_layout_passes=False),
        scratch_types=[
            pltpu.VMEM((_G * _DC,), jnp.float32),
            pltpu.VMEM((_R, 128), jnp.float32),
            pltpu.VMEM((_R, 128), jnp.float32),
            pltpu.VMEM((_R * 16,), jnp.float32),
            pltpu.VMEM((_R * 16,), jnp.float32),
            pltpu.VMEM((_R * 16,), jnp.int32),
            pltpu.VMEM((_R * 16,), jnp.int32),
            pltpu.SemaphoreType.DMA,
            pltpu.SemaphoreType.DMA,
        ],
    )


def _combine_body(p_ref, o_ref):
    p = p_ref[...]
    s0 = p[0, 0] + p[0, 1] + p[0, 2] + p[0, 3]
    s1 = p[1, 0] + p[1, 1] + p[1, 2] + p[1, 3]
    o_ref[...] = jnp.concatenate([s0, s1], axis=1)


def kernel(x, batch, W, b):
    n, d = x.shape
    npad = _NW * _R * (-(-n // (_NW * _R)))

    br = 10000
    logits, mz = pl.pallas_call(
        _logits_stats_body,
        grid=(n // br,),
        in_specs=[pl.BlockSpec((br, d), lambda i: (i, 0)),
                  pl.BlockSpec((d, 1), lambda i: (0, 0))],
        out_specs=[pl.BlockSpec((br, 1), lambda i: (i, 0)),
                   pl.BlockSpec((1, 2), lambda i: (0, 0))],
        out_shape=[jax.ShapeDtypeStruct((n, 1), jnp.float32),
                   jax.ShapeDtypeStruct((1, 2), jnp.float32)],
        scratch_shapes=[pltpu.SMEM((2,), jnp.float32)],
    )(x, W.reshape(d, 1))

    bs = 800
    nvb = n // bs

    def _alpha_body(l_ref, b_ref, mz_ref, a_ref, f_ref):
        i = pl.program_id(0)
        m = mz_ref[0, 0]
        z = mz_ref[0, 1]
        valid = (lax.broadcasted_iota(jnp.int32, (bs, 1), 0) + i * bs) < n
        a = jnp.where(valid, jnp.exp(l_ref[...] - m) / z, 0.0)
        a_ref[...] = jnp.broadcast_to(a, (bs, 16))
        fb = jnp.where(valid, b_ref[...], 0) * _DC
        f_ref[...] = fb + lax.broadcasted_iota(jnp.int32, (bs, 16), 1)

    a16, b16 = pl.pallas_call(
        _alpha_body,
        grid=(npad // bs,),
        in_specs=[
            pl.BlockSpec((bs, 1), lambda i: (jnp.minimum(i, nvb - 1), 0)),
            pl.BlockSpec((bs, 1), lambda i: (jnp.minimum(i, nvb - 1), 0)),
            pl.BlockSpec((1, 2), lambda i: (0, 0))],
        out_specs=[pl.BlockSpec((bs, 16), lambda i: (i, 0)),
                   pl.BlockSpec((bs, 16), lambda i: (i, 0))],
        out_shape=[jax.ShapeDtypeStruct((npad, 16), jnp.float32),
                   jax.ShapeDtypeStruct((npad, 16), jnp.int32)],
    )(logits, batch.astype(jnp.int32).reshape(n, 1), mz)

    zeros = jnp.zeros((_G * _DC,), jnp.float32)

    part = _make_pool(n, d, npad)(
        x, a16.reshape(npad * 16), b16.reshape(npad * 16), zeros)

    out = pl.pallas_call(
        _combine_body,
        grid=(_CG // 2,),
        in_specs=[pl.BlockSpec((2, _RG, _G, _DC), lambda i: (i, 0, 0, 0))],
        out_specs=pl.BlockSpec((_G, 2 * _DC), lambda i: (0, i)),
        out_shape=jax.ShapeDtypeStruct((_G, d), jnp.float32),
    )(part.reshape(_CG, _RG, _G, _DC))

    alpha = a16[:n, :1]
    return (out, alpha)
